# baseline (speedup 1.0000x reference)
import numpy as np
import jax
import jax.numpy as jnp
from jax import lax
from jax.experimental import pallas as pl
from jax.experimental.pallas import tpu as pltpu

_IM_H, _IM_W = 512, 512
_FEAT_H, _FEAT_W = 32, 32
_STRIDE = 16
_POS_THRES, _NEG_THRES = 0.7, 0.3
_N_SAMPLES = 256
_N_POS = _N_SAMPLES // 2
_A = _FEAT_H * _FEAT_W * 9
_R, _C = 72, 128
_INF_BITS = np.int32(0x7F800000)


def _make_anchors():
    ratios = [0.5, 1.0, 2.0]
    scales = [8.0, 16.0, 32.0]
    hs, ws = [], []
    for r in ratios:
        for s in scales:
            hs.append(_STRIDE * s * np.sqrt(r))
            ws.append(_STRIDE * s * np.sqrt(1.0 / r))
    hs = np.array(hs, dtype=np.float64)
    ws = np.array(ws, dtype=np.float64)
    sy = (np.arange(_FEAT_H) + 0.5) * _STRIDE
    sx = (np.arange(_FEAT_W) + 0.5) * _STRIDE
    cy, cx = np.meshgrid(sy, sx, indexing="ij")
    cy = cy.reshape(-1, 1)
    cx = cx.reshape(-1, 1)
    anchors = np.stack(
        [cy - 0.5 * hs, cx - 0.5 * ws, cy + 0.5 * hs, cx + 0.5 * ws], axis=-1
    )
    return anchors.reshape(-1, 4).astype(np.float32)


_ANCHORS = _make_anchors()
_AY1 = _ANCHORS[:, 0].reshape(_R, _C)
_AX1 = _ANCHORS[:, 1].reshape(_R, _C)
_AY2 = _ANCHORS[:, 2].reshape(_R, _C)
_AX2 = _ANCHORS[:, 3].reshape(_R, _C)
_INSIDE = (
    (_ANCHORS[:, 0] >= 0)
    & (_ANCHORS[:, 1] >= 0)
    & (_ANCHORS[:, 2] <= _IM_H)
    & (_ANCHORS[:, 3] <= _IM_W)
).reshape(_R, _C)
_AREA_A = ((_ANCHORS[:, 2] - _ANCHORS[:, 0]) * (_ANCHORS[:, 3] - _ANCHORS[:, 1])
           ).reshape(_R, _C)
_FAR = np.float32(-1.0e6)
_AY1_M = np.where(_INSIDE, _AY1, _FAR)
_AX1_M = np.where(_INSIDE, _AX1, _FAR)
_AY2_M = np.where(_INSIDE, _AY2, _FAR)
_AX2_M = np.where(_INSIDE, _AX2, _FAR)
_AREA_M = np.where(_INSIDE, _AREA_A, np.float32(1.0)).astype(np.float32)
_TF_ROT = ((13, 15, 26, 6), (17, 29, 16, 24))


def _tf_pair(k1, k2, x0, x1):
    ks = (np.uint32(k1), np.uint32(k2),
          np.uint32(np.uint32(k1) ^ np.uint32(k2) ^ np.uint32(0x1BD11BDA)))
    x0 = (x0 + ks[0]).astype(np.uint32)
    x1 = (x1 + ks[1]).astype(np.uint32)
    for r in range(5):
        for d in _TF_ROT[r % 2]:
            x0 = (x0 + x1).astype(np.uint32)
            x1 = ((x1 << np.uint32(d)) | (x1 >> np.uint32(32 - d))).astype(np.uint32)
            x1 = (x0 ^ x1).astype(np.uint32)
        x0 = (x0 + ks[(r + 1) % 3]).astype(np.uint32)
        x1 = (x1 + ks[(r + 2) % 3] + np.uint32(r + 1)).astype(np.uint32)
    return x0, x1


def _tf_uniform(k, n):
    b1, b2 = _tf_pair(k[0], k[1], np.zeros(n, np.uint32),
                      np.arange(n, dtype=np.uint32))
    bits = (b1 ^ b2).astype(np.uint32)
    f = ((bits >> np.uint32(9)) | np.uint32(0x3F800000)).view(np.float32)
    return np.maximum(np.float32(0.0), f - np.float32(1.0))


def _tf_fold_in(k, data):
    b0, b1 = _tf_pair(k[0], k[1],
                      np.array([(data >> 32) & 0xFFFFFFFF], np.uint32),
                      np.array([data & 0xFFFFFFFF], np.uint32))
    return b0[0], b1[0]


_SKEY = (np.uint32(0), np.uint32(42))
_RND_P = _tf_uniform(_SKEY, _A).reshape(_R, _C)
_RND_N = _tf_uniform(_tf_fold_in(_SKEY, 1), _A).reshape(_R, _C)
_M_P = np.round(_RND_P * np.float32(2.0 ** 23)).astype(np.int32)
_M_N = np.round(_RND_N * np.float32(2.0 ** 23)).astype(np.int32)
assert np.array_equal(_M_P.astype(np.float32) * np.float32(2.0 ** -23), _RND_P)
assert np.array_equal(_M_N.astype(np.float32) * np.float32(2.0 ** -23), _RND_N)
_M_PF = _M_P.astype(np.float32)
_M_NF = _M_N.astype(np.float32)
_M_TOP = float(1 << 23)
_M_BIG = float(1 << 24)
_UNROLL = 20


def _body(gt_ref, ay1_ref, ax1_ref, ay2_ref, ax2_ref, area_ref, inside_ref,
          rndp_ref, rndn_ref,
          dy_ref, dx_ref, dh_ref, dw_ref, lab_ref):
    inside = inside_ref[...] != 0

    neg_two = jnp.full((_R, _C), -2.0, dtype=jnp.float32)
    zero = jnp.zeros((_R, _C), dtype=jnp.float32)
    zero_i = jnp.zeros((_R, _C), dtype=jnp.int32)

    def one_gt(j, curmax, bestj, eq):
        gy1 = gt_ref[j, 0]
        gx1 = gt_ref[j, 1]
        gy2 = gt_ref[j, 2]
        gx2 = gt_ref[j, 3]
        tl_y = jnp.maximum(ay1_ref[...], gy1)
        tl_x = jnp.maximum(ax1_ref[...], gx1)
        br_y = jnp.minimum(ay2_ref[...], gy2)
        br_x = jnp.minimum(ax2_ref[...], gx2)
        h = jnp.maximum(br_y - tl_y, 0.0)
        w = jnp.maximum(br_x - tl_x, 0.0)
        inter = h * w
        area_g = (gy2 - gy1) * (gx2 - gx1)
        iou_m = inter / ((area_ref[...] + area_g) - inter)
        gmax = jnp.max(iou_m)
        eq = jnp.where(iou_m == gmax, 1, eq)
        upd = iou_m > curmax
        curmax = jnp.where(upd, iou_m, curmax)
        bestj = jnp.where(upd, j, bestj)
        return curmax, bestj, eq

    def gt_step(jj, carry):
        curmax, bestj, eq = carry
        j0 = jj * _UNROLL
        for t in range(_UNROLL):
            curmax, bestj, eq = one_gt(j0 + t, curmax, bestj, eq)
        return curmax, bestj, eq

    curmax, bestj, eq = lax.fori_loop(
        0, 100 // _UNROLL, gt_step, (neg_two, zero_i, zero_i))

    def exp_step(jj, carry):
        by1, bx1, by2, bx2 = carry
        for t in range(4):
            j = jj * 4 + t
            sel = bestj == j
            by1 = jnp.where(sel, gt_ref[j, 0], by1)
            bx1 = jnp.where(sel, gt_ref[j, 1], bx1)
            by2 = jnp.where(sel, gt_ref[j, 2], by2)
            bx2 = jnp.where(sel, gt_ref[j, 3], bx2)
        return by1, bx1, by2, bx2

    by1, bx1, by2, bx2 = lax.fori_loop(
        0, 25, exp_step, (zero, zero, zero, zero))

    bh = by2 - by1
    bw = bx2 - bx1
    bcy = by1 + 0.5 * bh
    bcx = bx1 + 0.5 * bw
    eps = jnp.float32(np.finfo(np.float32).eps)
    ay1 = ay1_ref[...]
    ax1 = ax1_ref[...]
    ay2 = ay2_ref[...]
    ax2 = ax2_ref[...]
    ah = jnp.maximum(ay2 - ay1, eps)
    aw = jnp.maximum(ax2 - ax1, eps)
    acy = ay1 + 0.5 * (ay2 - ay1)
    acx = ax1 + 0.5 * (ax2 - ax1)
    dy = (bcy - acy) / ah
    dx = (bcx - acx) / aw
    dh = jnp.log(bh / ah)
    dw = jnp.log(bw / aw)
    dy_ref[...] = jnp.where(inside, dy, 0.0)
    dx_ref[...] = jnp.where(inside, dx, 0.0)
    dh_ref[...] = jnp.where(inside, dh, 0.0)
    dw_ref[...] = jnp.where(inside, dw, 0.0)

    labels = jnp.full((_R, _C), -1, dtype=jnp.int32)
    labels = jnp.where(inside & (curmax < _NEG_THRES), 0, labels)
    labels = jnp.where(inside & (eq != 0), 1, labels)
    labels = jnp.where(inside & (curmax >= _POS_THRES), 1, labels)

    pos_mask = labels == 1
    neg_mask = labels == 0
    def count_vec(pred):
        ind = jnp.where(pred, 1.0, 0.0).astype(jnp.float32)
        psum = jnp.sum(ind.reshape(9, 8, _C), axis=0)
        row = jnp.sum(psum, axis=0, keepdims=True)
        for s in (1, 2, 4, 8, 16, 32, 64):
            row = row + pltpu.roll(row, s, axis=1)
        return row

    pos_count = count_vec(pos_mask)
    neg_count = count_vec(neg_mask)
    m_p = jnp.where(pos_mask, rndp_ref[...], _M_BIG)
    m_n = jnp.where(neg_mask, rndn_ref[...], _M_BIG)
    n_pos_f = jnp.float32(_N_POS)
    n_neg = jnp.where(pos_count < n_pos_f, pos_count, n_pos_f)
    k_p = jnp.full((1, _C), _N_POS, dtype=jnp.float32)
    k_n = jnp.maximum(n_neg, 1.0)

    def level(_, c):
        lo_p, hi_p, lo_n, hi_n = c

        def probe(m_plane, lo, hi, k):
            q = jnp.floor((hi - lo) * 0.25)
            m1 = lo + q
            m2 = lo + 2.0 * q
            m3 = lo + 3.0 * q
            b1 = count_vec(m_plane <= m1) >= k
            b2 = count_vec(m_plane <= m2) >= k
            b3 = count_vec(m_plane <= m3) >= k
            hi = jnp.where(b1, m1, jnp.where(b2, m2, jnp.where(b3, m3, hi)))
            lo = jnp.where(b1, lo,
                           jnp.where(b2, m1 + 1.0,
                                     jnp.where(b3, m2 + 1.0, m3 + 1.0)))
            return lo, hi

        lo_p, hi_p = probe(m_p, lo_p, hi_p, k_p)
        lo_n, hi_n = probe(m_n, lo_n, hi_n, k_n)
        return lo_p, hi_p, lo_n, hi_n

    z = jnp.zeros((1, _C), dtype=jnp.float32)
    top = jnp.full((1, _C), _M_TOP, dtype=jnp.float32)
    lo_p, _, lo_n, _ = lax.fori_loop(0, 14, level, (z, top, z, top))

    kill_p = (pos_count > n_pos_f) & pos_mask & (m_p > lo_p)
    labels = jnp.where(kill_p, -1, labels)
    over_neg = neg_count > n_neg
    kill_n = over_neg & (
        ((n_neg > 0) & neg_mask & (m_n > lo_n)) | ((n_neg == 0) & neg_mask)
    )
    labels = jnp.where(kill_n, -1, labels)
    lab_ref[...] = labels


def kernel(image, feature_map, gt_boxes):
    del image, feature_map
    plane = jax.ShapeDtypeStruct((_R, _C), jnp.float32)
    outs = pl.pallas_call(
        _body,
        out_shape=(
            plane, plane, plane, plane,
            jax.ShapeDtypeStruct((_R, _C), jnp.int32),
        ),
        in_specs=[
            pl.BlockSpec(memory_space=pltpu.SMEM),
            pl.BlockSpec(memory_space=pltpu.VMEM),
            pl.BlockSpec(memory_space=pltpu.VMEM),
            pl.BlockSpec(memory_space=pltpu.VMEM),
            pl.BlockSpec(memory_space=pltpu.VMEM),
            pl.BlockSpec(memory_space=pltpu.VMEM),
            pl.BlockSpec(memory_space=pltpu.VMEM),
            pl.BlockSpec(memory_space=pltpu.VMEM),
            pl.BlockSpec(memory_space=pltpu.VMEM),
        ],
    )(
        gt_boxes,
        jnp.asarray(_AY1_M), jnp.asarray(_AX1_M),
        jnp.asarray(_AY2_M), jnp.asarray(_AX2_M),
        jnp.asarray(_AREA_M),
        jnp.asarray(_INSIDE.astype(np.int32)),
        jnp.asarray(_M_PF), jnp.asarray(_M_NF),
    )
    dy, dx, dh, dw, labels = outs
    locs = jnp.stack(
        [dy.reshape(_A), dx.reshape(_A), dh.reshape(_A), dw.reshape(_A)],
        axis=1,
    )
    return locs, labels.reshape(_A), jnp.asarray(_ANCHORS)


# speedup vs baseline: 1.3932x; 1.3932x over previous
import numpy as np
import jax
import jax.numpy as jnp
from jax import lax
from jax.experimental import pallas as pl
from jax.experimental.pallas import tpu as pltpu

_IM_H, _IM_W = 512, 512
_FEAT_H, _FEAT_W = 32, 32
_STRIDE = 16
_POS_THRES, _NEG_THRES = 0.7, 0.3
_N_SAMPLES = 256
_N_POS = _N_SAMPLES // 2
_A = _FEAT_H * _FEAT_W * 9
_R, _C = 72, 128
_INF_BITS = np.int32(0x7F800000)


def _make_anchors():
    ratios = [0.5, 1.0, 2.0]
    scales = [8.0, 16.0, 32.0]
    hs, ws = [], []
    for r in ratios:
        for s in scales:
            hs.append(_STRIDE * s * np.sqrt(r))
            ws.append(_STRIDE * s * np.sqrt(1.0 / r))
    hs = np.array(hs, dtype=np.float64)
    ws = np.array(ws, dtype=np.float64)
    sy = (np.arange(_FEAT_H) + 0.5) * _STRIDE
    sx = (np.arange(_FEAT_W) + 0.5) * _STRIDE
    cy, cx = np.meshgrid(sy, sx, indexing="ij")
    cy = cy.reshape(-1, 1)
    cx = cx.reshape(-1, 1)
    anchors = np.stack(
        [cy - 0.5 * hs, cx - 0.5 * ws, cy + 0.5 * hs, cx + 0.5 * ws], axis=-1
    )
    return anchors.reshape(-1, 4).astype(np.float32)


_ANCHORS = _make_anchors()
_AY1 = _ANCHORS[:, 0].reshape(_R, _C)
_AX1 = _ANCHORS[:, 1].reshape(_R, _C)
_AY2 = _ANCHORS[:, 2].reshape(_R, _C)
_AX2 = _ANCHORS[:, 3].reshape(_R, _C)
_INSIDE = (
    (_ANCHORS[:, 0] >= 0)
    & (_ANCHORS[:, 1] >= 0)
    & (_ANCHORS[:, 2] <= _IM_H)
    & (_ANCHORS[:, 3] <= _IM_W)
).reshape(_R, _C)
_AREA_A = ((_ANCHORS[:, 2] - _ANCHORS[:, 0]) * (_ANCHORS[:, 3] - _ANCHORS[:, 1])
           ).reshape(_R, _C)
_FAR = np.float32(-1.0e6)
_AY1_M = np.where(_INSIDE, _AY1, _FAR)
_AX1_M = np.where(_INSIDE, _AX1, _FAR)
_AY2_M = np.where(_INSIDE, _AY2, _FAR)
_AX2_M = np.where(_INSIDE, _AX2, _FAR)
_AREA_M = np.where(_INSIDE, _AREA_A, np.float32(1.0)).astype(np.float32)
_TF_ROT = ((13, 15, 26, 6), (17, 29, 16, 24))


def _tf_pair(k1, k2, x0, x1):
    ks = (np.uint32(k1), np.uint32(k2),
          np.uint32(np.uint32(k1) ^ np.uint32(k2) ^ np.uint32(0x1BD11BDA)))
    x0 = (x0 + ks[0]).astype(np.uint32)
    x1 = (x1 + ks[1]).astype(np.uint32)
    for r in range(5):
        for d in _TF_ROT[r % 2]:
            x0 = (x0 + x1).astype(np.uint32)
            x1 = ((x1 << np.uint32(d)) | (x1 >> np.uint32(32 - d))).astype(np.uint32)
            x1 = (x0 ^ x1).astype(np.uint32)
        x0 = (x0 + ks[(r + 1) % 3]).astype(np.uint32)
        x1 = (x1 + ks[(r + 2) % 3] + np.uint32(r + 1)).astype(np.uint32)
    return x0, x1


def _tf_uniform(k, n):
    b1, b2 = _tf_pair(k[0], k[1], np.zeros(n, np.uint32),
                      np.arange(n, dtype=np.uint32))
    bits = (b1 ^ b2).astype(np.uint32)
    f = ((bits >> np.uint32(9)) | np.uint32(0x3F800000)).view(np.float32)
    return np.maximum(np.float32(0.0), f - np.float32(1.0))


def _tf_fold_in(k, data):
    b0, b1 = _tf_pair(k[0], k[1],
                      np.array([(data >> 32) & 0xFFFFFFFF], np.uint32),
                      np.array([data & 0xFFFFFFFF], np.uint32))
    return b0[0], b1[0]


_SKEY = (np.uint32(0), np.uint32(42))
_RND_P = _tf_uniform(_SKEY, _A).reshape(_R, _C)
_RND_N = _tf_uniform(_tf_fold_in(_SKEY, 1), _A).reshape(_R, _C)
_M_P = np.round(_RND_P * np.float32(2.0 ** 23)).astype(np.int32)
_M_N = np.round(_RND_N * np.float32(2.0 ** 23)).astype(np.int32)
assert np.array_equal(_M_P.astype(np.float32) * np.float32(2.0 ** -23), _RND_P)
assert np.array_equal(_M_N.astype(np.float32) * np.float32(2.0 ** -23), _RND_N)
_M_PF = _M_P.astype(np.float32)
_M_NF = _M_N.astype(np.float32)
_M_TOP = float(1 << 23)
_M_BIG = float(1 << 24)
_UNROLL = 20


def _body(gt_ref, ay1_ref, ax1_ref, ay2_ref, ax2_ref, area_ref, inside_ref,
          rndp_ref, rndn_ref,
          dy_ref, dx_ref, dh_ref, dw_ref, lab_ref):
    inside = inside_ref[...] != 0

    neg_two = jnp.full((_R, _C), -2.0, dtype=jnp.float32)
    zero = jnp.zeros((_R, _C), dtype=jnp.float32)
    zero_i = jnp.zeros((_R, _C), dtype=jnp.int32)

    def one_gt(j, curmax, bestj, eq):
        gy1 = gt_ref[j, 0]
        gx1 = gt_ref[j, 1]
        gy2 = gt_ref[j, 2]
        gx2 = gt_ref[j, 3]
        tl_y = jnp.maximum(ay1_ref[...], gy1)
        tl_x = jnp.maximum(ax1_ref[...], gx1)
        br_y = jnp.minimum(ay2_ref[...], gy2)
        br_x = jnp.minimum(ax2_ref[...], gx2)
        h = jnp.maximum(br_y - tl_y, 0.0)
        w = jnp.maximum(br_x - tl_x, 0.0)
        inter = h * w
        area_g = (gy2 - gy1) * (gx2 - gx1)
        iou_m = inter / ((area_ref[...] + area_g) - inter)
        gmax = jnp.max(iou_m)
        eq = jnp.where(iou_m == gmax, 1, eq)
        upd = iou_m > curmax
        curmax = jnp.where(upd, iou_m, curmax)
        bestj = jnp.where(upd, j, bestj)
        return curmax, bestj, eq

    def gt_step(jj, carry):
        curmax, bestj, eq = carry
        j0 = jj * _UNROLL
        for t in range(_UNROLL):
            curmax, bestj, eq = one_gt(j0 + t, curmax, bestj, eq)
        return curmax, bestj, eq

    curmax, bestj, eq = lax.fori_loop(
        0, 100 // _UNROLL, gt_step, (neg_two, zero_i, zero_i))

    def exp_step(jj, carry):
        by1, bx1, by2, bx2 = carry
        for t in range(10):
            j = jj * 10 + t
            sel = bestj == j
            by1 = jnp.where(sel, gt_ref[j, 0], by1)
            bx1 = jnp.where(sel, gt_ref[j, 1], bx1)
            by2 = jnp.where(sel, gt_ref[j, 2], by2)
            bx2 = jnp.where(sel, gt_ref[j, 3], bx2)
        return by1, bx1, by2, bx2

    by1, bx1, by2, bx2 = lax.fori_loop(
        0, 10, exp_step, (zero, zero, zero, zero))

    bh = by2 - by1
    bw = bx2 - bx1
    bcy = by1 + 0.5 * bh
    bcx = bx1 + 0.5 * bw
    eps = jnp.float32(np.finfo(np.float32).eps)
    ay1 = ay1_ref[...]
    ax1 = ax1_ref[...]
    ay2 = ay2_ref[...]
    ax2 = ax2_ref[...]
    ah = jnp.maximum(ay2 - ay1, eps)
    aw = jnp.maximum(ax2 - ax1, eps)
    acy = ay1 + 0.5 * (ay2 - ay1)
    acx = ax1 + 0.5 * (ax2 - ax1)
    dy = (bcy - acy) / ah
    dx = (bcx - acx) / aw
    dh = jnp.log(bh / ah)
    dw = jnp.log(bw / aw)
    dy_ref[...] = jnp.where(inside, dy, 0.0)
    dx_ref[...] = jnp.where(inside, dx, 0.0)
    dh_ref[...] = jnp.where(inside, dh, 0.0)
    dw_ref[...] = jnp.where(inside, dw, 0.0)

    labels = jnp.full((_R, _C), -1, dtype=jnp.int32)
    labels = jnp.where(inside & (curmax < _NEG_THRES), 0, labels)
    labels = jnp.where(inside & (eq != 0), 1, labels)
    labels = jnp.where(inside & (curmax >= _POS_THRES), 1, labels)

    pos_mask = labels == 1
    neg_mask = labels == 0
    ones_ll = jnp.ones((_C, _C), dtype=jnp.float32)

    def count_vec(pred):
        ind = jnp.where(pred, 1.0, 0.0).astype(jnp.float32)
        psum = jnp.sum(ind.reshape(9, 8, _C), axis=0)
        lane_tot = jnp.dot(psum, ones_ll)
        return jnp.sum(lane_tot, axis=0, keepdims=True)

    pos_count = count_vec(pos_mask)
    neg_count = count_vec(neg_mask)
    m_p = jnp.where(pos_mask, rndp_ref[...], _M_BIG)
    m_n = jnp.where(neg_mask, rndn_ref[...], _M_BIG)
    n_pos_f = jnp.float32(_N_POS)
    n_neg = jnp.where(pos_count < n_pos_f, pos_count, n_pos_f)
    k_p = jnp.full((1, _C), _N_POS, dtype=jnp.float32)
    k_n = jnp.maximum(n_neg, 1.0)

    def level(_, c):
        lo_p, hi_p, lo_n, hi_n = c

        def probe(m_plane, lo, hi, k):
            q = jnp.floor((hi - lo) * 0.25)
            m1 = lo + q
            m2 = lo + 2.0 * q
            m3 = lo + 3.0 * q
            b1 = count_vec(m_plane <= m1) >= k
            b2 = count_vec(m_plane <= m2) >= k
            b3 = count_vec(m_plane <= m3) >= k
            hi = jnp.where(b1, m1, jnp.where(b2, m2, jnp.where(b3, m3, hi)))
            lo = jnp.where(b1, lo,
                           jnp.where(b2, m1 + 1.0,
                                     jnp.where(b3, m2 + 1.0, m3 + 1.0)))
            return lo, hi

        lo_p, hi_p = probe(m_p, lo_p, hi_p, k_p)
        lo_n, hi_n = probe(m_n, lo_n, hi_n, k_n)
        return lo_p, hi_p, lo_n, hi_n

    z = jnp.zeros((1, _C), dtype=jnp.float32)
    top = jnp.full((1, _C), _M_TOP, dtype=jnp.float32)
    lo_p, _, lo_n, _ = lax.fori_loop(0, 14, level, (z, top, z, top))

    kill_p = (pos_count > n_pos_f) & pos_mask & (m_p > lo_p)
    labels = jnp.where(kill_p, -1, labels)
    over_neg = neg_count > n_neg
    kill_n = over_neg & (
        ((n_neg > 0) & neg_mask & (m_n > lo_n)) | ((n_neg == 0) & neg_mask)
    )
    labels = jnp.where(kill_n, -1, labels)
    lab_ref[...] = labels


def kernel(image, feature_map, gt_boxes):
    del image, feature_map
    plane = jax.ShapeDtypeStruct((_R, _C), jnp.float32)
    outs = pl.pallas_call(
        _body,
        out_shape=(
            plane, plane, plane, plane,
            jax.ShapeDtypeStruct((_R, _C), jnp.int32),
        ),
        in_specs=[
            pl.BlockSpec(memory_space=pltpu.SMEM),
            pl.BlockSpec(memory_space=pltpu.VMEM),
            pl.BlockSpec(memory_space=pltpu.VMEM),
            pl.BlockSpec(memory_space=pltpu.VMEM),
            pl.BlockSpec(memory_space=pltpu.VMEM),
            pl.BlockSpec(memory_space=pltpu.VMEM),
            pl.BlockSpec(memory_space=pltpu.VMEM),
            pl.BlockSpec(memory_space=pltpu.VMEM),
            pl.BlockSpec(memory_space=pltpu.VMEM),
        ],
    )(
        gt_boxes,
        jnp.asarray(_AY1_M), jnp.asarray(_AX1_M),
        jnp.asarray(_AY2_M), jnp.asarray(_AX2_M),
        jnp.asarray(_AREA_M),
        jnp.asarray(_INSIDE.astype(np.int32)),
        jnp.asarray(_M_PF), jnp.asarray(_M_NF),
    )
    dy, dx, dh, dw, labels = outs
    locs = jnp.stack(
        [dy.reshape(_A), dx.reshape(_A), dh.reshape(_A), dw.reshape(_A)],
        axis=1,
    )
    return locs, labels.reshape(_A), jnp.asarray(_ANCHORS)


# speedup vs baseline: 1.4028x; 1.0069x over previous
import numpy as np
import jax
import jax.numpy as jnp
from jax import lax
from jax.experimental import pallas as pl
from jax.experimental.pallas import tpu as pltpu

_IM_H, _IM_W = 512, 512
_FEAT_H, _FEAT_W = 32, 32
_STRIDE = 16
_POS_THRES, _NEG_THRES = 0.7, 0.3
_N_SAMPLES = 256
_N_POS = _N_SAMPLES // 2
_A = _FEAT_H * _FEAT_W * 9
_R, _C = 72, 128


def _make_anchors():
    ratios = [0.5, 1.0, 2.0]
    scales = [8.0, 16.0, 32.0]
    hs, ws = [], []
    for r in ratios:
        for s in scales:
            hs.append(_STRIDE * s * np.sqrt(r))
            ws.append(_STRIDE * s * np.sqrt(1.0 / r))
    hs = np.array(hs, dtype=np.float64)
    ws = np.array(ws, dtype=np.float64)
    sy = (np.arange(_FEAT_H) + 0.5) * _STRIDE
    sx = (np.arange(_FEAT_W) + 0.5) * _STRIDE
    cy, cx = np.meshgrid(sy, sx, indexing="ij")
    cy = cy.reshape(-1, 1)
    cx = cx.reshape(-1, 1)
    anchors = np.stack(
        [cy - 0.5 * hs, cx - 0.5 * ws, cy + 0.5 * hs, cx + 0.5 * ws], axis=-1
    )
    return anchors.reshape(-1, 4).astype(np.float32)


_ANCHORS = _make_anchors()
_AY1 = _ANCHORS[:, 0].reshape(_R, _C)
_AX1 = _ANCHORS[:, 1].reshape(_R, _C)
_AY2 = _ANCHORS[:, 2].reshape(_R, _C)
_AX2 = _ANCHORS[:, 3].reshape(_R, _C)
_INSIDE = (
    (_ANCHORS[:, 0] >= 0)
    & (_ANCHORS[:, 1] >= 0)
    & (_ANCHORS[:, 2] <= _IM_H)
    & (_ANCHORS[:, 3] <= _IM_W)
).reshape(_R, _C)
_AREA_A = ((_ANCHORS[:, 2] - _ANCHORS[:, 0]) * (_ANCHORS[:, 3] - _ANCHORS[:, 1])
           ).reshape(_R, _C)
_FAR = np.float32(-1.0e6)
_AY1_M = np.where(_INSIDE, _AY1, _FAR)
_AX1_M = np.where(_INSIDE, _AX1, _FAR)
_AY2_M = np.where(_INSIDE, _AY2, _FAR)
_AX2_M = np.where(_INSIDE, _AX2, _FAR)
_AREA_M = np.where(_INSIDE, _AREA_A, np.float32(1.0)).astype(np.float32)
_TF_ROT = ((13, 15, 26, 6), (17, 29, 16, 24))


def _tf_pair(k1, k2, x0, x1):
    ks = (np.uint32(k1), np.uint32(k2),
          np.uint32(np.uint32(k1) ^ np.uint32(k2) ^ np.uint32(0x1BD11BDA)))
    x0 = (x0 + ks[0]).astype(np.uint32)
    x1 = (x1 + ks[1]).astype(np.uint32)
    for r in range(5):
        for d in _TF_ROT[r % 2]:
            x0 = (x0 + x1).astype(np.uint32)
            x1 = ((x1 << np.uint32(d)) | (x1 >> np.uint32(32 - d))).astype(np.uint32)
            x1 = (x0 ^ x1).astype(np.uint32)
        x0 = (x0 + ks[(r + 1) % 3]).astype(np.uint32)
        x1 = (x1 + ks[(r + 2) % 3] + np.uint32(r + 1)).astype(np.uint32)
    return x0, x1


def _tf_uniform(k, n):
    b1, b2 = _tf_pair(k[0], k[1], np.zeros(n, np.uint32),
                      np.arange(n, dtype=np.uint32))
    bits = (b1 ^ b2).astype(np.uint32)
    f = ((bits >> np.uint32(9)) | np.uint32(0x3F800000)).view(np.float32)
    return np.maximum(np.float32(0.0), f - np.float32(1.0))


def _tf_fold_in(k, data):
    b0, b1 = _tf_pair(k[0], k[1],
                      np.array([(data >> 32) & 0xFFFFFFFF], np.uint32),
                      np.array([data & 0xFFFFFFFF], np.uint32))
    return b0[0], b1[0]


_SKEY = (np.uint32(0), np.uint32(42))
_RND_P = _tf_uniform(_SKEY, _A).reshape(_R, _C)
_RND_N = _tf_uniform(_tf_fold_in(_SKEY, 1), _A).reshape(_R, _C)
_M_P = np.round(_RND_P * np.float32(2.0 ** 23)).astype(np.int32)
_M_N = np.round(_RND_N * np.float32(2.0 ** 23)).astype(np.int32)
assert np.array_equal(_M_P.astype(np.float32) * np.float32(2.0 ** -23), _RND_P)
assert np.array_equal(_M_N.astype(np.float32) * np.float32(2.0 ** -23), _RND_N)
_M_PF = _M_P.astype(np.float32)
_M_NF = _M_N.astype(np.float32)
_M_TOP = float(1 << 23)
_M_BIG = float(1 << 24)
_UNROLL = 20


def _body(gt_ref, ay1_ref, ax1_ref, ay2_ref, ax2_ref, area_ref, inside_ref,
          rndp_ref, rndn_ref,
          dy_ref, dx_ref, dh_ref, dw_ref, lab_ref):
    inside = inside_ref[...] != 0

    neg_two = jnp.full((_R, _C), -2.0, dtype=jnp.float32)
    zero = jnp.zeros((_R, _C), dtype=jnp.float32)
    zero_i = jnp.zeros((_R, _C), dtype=jnp.int32)

    def one_gt(j, curmax, bestj, eq):
        gy1 = gt_ref[j, 0]
        gx1 = gt_ref[j, 1]
        gy2 = gt_ref[j, 2]
        gx2 = gt_ref[j, 3]
        tl_y = jnp.maximum(ay1_ref[...], gy1)
        tl_x = jnp.maximum(ax1_ref[...], gx1)
        br_y = jnp.minimum(ay2_ref[...], gy2)
        br_x = jnp.minimum(ax2_ref[...], gx2)
        h = jnp.maximum(br_y - tl_y, 0.0)
        w = jnp.maximum(br_x - tl_x, 0.0)
        inter = h * w
        area_g = (gy2 - gy1) * (gx2 - gx1)
        iou_m = inter / ((area_ref[...] + area_g) - inter)
        gmax = jnp.max(iou_m)
        eq = jnp.where(iou_m == gmax, 1, eq)
        upd = iou_m > curmax
        curmax = jnp.where(upd, iou_m, curmax)
        bestj = jnp.where(upd, j, bestj)
        return curmax, bestj, eq

    def gt_step(jj, carry):
        curmax, bestj, eq = carry
        j0 = jj * _UNROLL
        for t in range(_UNROLL):
            curmax, bestj, eq = one_gt(j0 + t, curmax, bestj, eq)
        return curmax, bestj, eq

    curmax, bestj, eq = lax.fori_loop(
        0, 100 // _UNROLL, gt_step, (neg_two, zero_i, zero_i))

    def exp_step(jj, carry):
        by1, bx1, by2, bx2 = carry
        for t in range(10):
            j = jj * 10 + t
            sel = bestj == j
            by1 = jnp.where(sel, gt_ref[j, 0], by1)
            bx1 = jnp.where(sel, gt_ref[j, 1], bx1)
            by2 = jnp.where(sel, gt_ref[j, 2], by2)
            bx2 = jnp.where(sel, gt_ref[j, 3], bx2)
        return by1, bx1, by2, bx2

    by1, bx1, by2, bx2 = lax.fori_loop(
        0, 10, exp_step, (zero, zero, zero, zero))

    bh = by2 - by1
    bw = bx2 - bx1
    bcy = by1 + 0.5 * bh
    bcx = bx1 + 0.5 * bw
    eps = jnp.float32(np.finfo(np.float32).eps)
    ay1 = ay1_ref[...]
    ax1 = ax1_ref[...]
    ay2 = ay2_ref[...]
    ax2 = ax2_ref[...]
    ah = jnp.maximum(ay2 - ay1, eps)
    aw = jnp.maximum(ax2 - ax1, eps)
    acy = ay1 + 0.5 * (ay2 - ay1)
    acx = ax1 + 0.5 * (ax2 - ax1)
    dy = (bcy - acy) / ah
    dx = (bcx - acx) / aw
    dh = jnp.log(bh / ah)
    dw = jnp.log(bw / aw)
    dy_ref[...] = jnp.where(inside, dy, 0.0)
    dx_ref[...] = jnp.where(inside, dx, 0.0)
    dh_ref[...] = jnp.where(inside, dh, 0.0)
    dw_ref[...] = jnp.where(inside, dw, 0.0)

    labels = jnp.full((_R, _C), -1, dtype=jnp.int32)
    labels = jnp.where(inside & (curmax < _NEG_THRES), 0, labels)
    labels = jnp.where(inside & (eq != 0), 1, labels)
    labels = jnp.where(inside & (curmax >= _POS_THRES), 1, labels)

    pos_mask = labels == 1
    neg_mask = labels == 0
    ones_ll = jnp.ones((_C, _C), dtype=jnp.float32)

    def count_vec(pred):
        ind = jnp.where(pred, 1.0, 0.0).astype(jnp.float32)
        psum = jnp.sum(ind.reshape(9, 8, _C), axis=0)
        lane_tot = jnp.dot(psum, ones_ll)
        return jnp.sum(lane_tot, axis=0, keepdims=True)

    pos_count = count_vec(pos_mask)
    neg_count = count_vec(neg_mask)
    m_p = jnp.where(pos_mask, rndp_ref[...], _M_BIG)
    m_n = jnp.where(neg_mask, rndn_ref[...], _M_BIG)
    n_pos_f = jnp.float32(_N_POS)
    n_neg = jnp.where(pos_count < n_pos_f, pos_count, n_pos_f)
    k_p = jnp.full((1, _C), _N_POS, dtype=jnp.float32)
    k_n = jnp.maximum(n_neg, 1.0)

    def level(_, c):
        lo_p, hi_p, lo_n, hi_n = c

        def probe(m_plane, lo, hi, k):
            q = jnp.floor((hi - lo) * 0.25)
            m1 = lo + q
            m2 = lo + 2.0 * q
            m3 = lo + 3.0 * q
            b1 = count_vec(m_plane <= m1) >= k
            b2 = count_vec(m_plane <= m2) >= k
            b3 = count_vec(m_plane <= m3) >= k
            hi = jnp.where(b1, m1, jnp.where(b2, m2, jnp.where(b3, m3, hi)))
            lo = jnp.where(b1, lo,
                           jnp.where(b2, m1 + 1.0,
                                     jnp.where(b3, m2 + 1.0, m3 + 1.0)))
            return lo, hi

        lo_p, hi_p = probe(m_p, lo_p, hi_p, k_p)
        lo_n, hi_n = probe(m_n, lo_n, hi_n, k_n)
        return lo_p, hi_p, lo_n, hi_n

    z = jnp.zeros((1, _C), dtype=jnp.float32)
    top = jnp.full((1, _C), _M_TOP, dtype=jnp.float32)
    lo_p, _, lo_n, _ = lax.fori_loop(0, 14, level, (z, top, z, top))

    kill_p = (pos_count > n_pos_f) & pos_mask & (m_p > lo_p)
    labels = jnp.where(kill_p, -1, labels)
    over_neg = neg_count > n_neg
    kill_n = over_neg & (
        ((n_neg > 0) & neg_mask & (m_n > lo_n)) | ((n_neg == 0) & neg_mask)
    )
    labels = jnp.where(kill_n, -1, labels)
    lab_ref[...] = labels


def kernel(image, feature_map, gt_boxes):
    del image, feature_map
    plane = jax.ShapeDtypeStruct((_R, _C), jnp.float32)
    outs = pl.pallas_call(
        _body,
        out_shape=(
            plane, plane, plane, plane,
            jax.ShapeDtypeStruct((_R, _C), jnp.int32),
        ),
        in_specs=[
            pl.BlockSpec(memory_space=pltpu.SMEM),
            pl.BlockSpec(memory_space=pltpu.VMEM),
            pl.BlockSpec(memory_space=pltpu.VMEM),
            pl.BlockSpec(memory_space=pltpu.VMEM),
            pl.BlockSpec(memory_space=pltpu.VMEM),
            pl.BlockSpec(memory_space=pltpu.VMEM),
            pl.BlockSpec(memory_space=pltpu.VMEM),
            pl.BlockSpec(memory_space=pltpu.VMEM),
            pl.BlockSpec(memory_space=pltpu.VMEM),
        ],
    )(
        gt_boxes,
        jnp.asarray(_AY1_M), jnp.asarray(_AX1_M),
        jnp.asarray(_AY2_M), jnp.asarray(_AX2_M),
        jnp.asarray(_AREA_M),
        jnp.asarray(_INSIDE.astype(np.int32)),
        jnp.asarray(_M_PF), jnp.asarray(_M_NF),
    )
    dy, dx, dh, dw, labels = outs
    locs = jnp.stack(
        [dy.reshape(_A), dx.reshape(_A), dh.reshape(_A), dw.reshape(_A)],
        axis=1,
    )
    return locs, labels.reshape(_A), jnp.asarray(_ANCHORS)


# speedup vs baseline: 1.4325x; 1.0212x over previous
import numpy as np
import jax
import jax.numpy as jnp
from jax import lax
from jax.experimental import pallas as pl
from jax.experimental.pallas import tpu as pltpu

_IM_H, _IM_W = 512, 512
_FEAT_H, _FEAT_W = 32, 32
_STRIDE = 16
_POS_THRES, _NEG_THRES = 0.7, 0.3
_N_SAMPLES = 256
_N_POS = _N_SAMPLES // 2
_A = _FEAT_H * _FEAT_W * 9
_R, _C = 72, 128


def _make_anchors():
    ratios = [0.5, 1.0, 2.0]
    scales = [8.0, 16.0, 32.0]
    hs, ws = [], []
    for r in ratios:
        for s in scales:
            hs.append(_STRIDE * s * np.sqrt(r))
            ws.append(_STRIDE * s * np.sqrt(1.0 / r))
    hs = np.array(hs, dtype=np.float64)
    ws = np.array(ws, dtype=np.float64)
    sy = (np.arange(_FEAT_H) + 0.5) * _STRIDE
    sx = (np.arange(_FEAT_W) + 0.5) * _STRIDE
    cy, cx = np.meshgrid(sy, sx, indexing="ij")
    cy = cy.reshape(-1, 1)
    cx = cx.reshape(-1, 1)
    anchors = np.stack(
        [cy - 0.5 * hs, cx - 0.5 * ws, cy + 0.5 * hs, cx + 0.5 * ws], axis=-1
    )
    return anchors.reshape(-1, 4).astype(np.float32)


_ANCHORS = _make_anchors()
_AY1 = _ANCHORS[:, 0].reshape(_R, _C)
_AX1 = _ANCHORS[:, 1].reshape(_R, _C)
_AY2 = _ANCHORS[:, 2].reshape(_R, _C)
_AX2 = _ANCHORS[:, 3].reshape(_R, _C)
_INSIDE = (
    (_ANCHORS[:, 0] >= 0)
    & (_ANCHORS[:, 1] >= 0)
    & (_ANCHORS[:, 2] <= _IM_H)
    & (_ANCHORS[:, 3] <= _IM_W)
).reshape(_R, _C)
_AREA_A = ((_ANCHORS[:, 2] - _ANCHORS[:, 0]) * (_ANCHORS[:, 3] - _ANCHORS[:, 1])
           ).reshape(_R, _C)
_FAR = np.float32(-1.0e6)
_AY1_M = np.where(_INSIDE, _AY1, _FAR)
_AX1_M = np.where(_INSIDE, _AX1, _FAR)
_AY2_M = np.where(_INSIDE, _AY2, _FAR)
_AX2_M = np.where(_INSIDE, _AX2, _FAR)
_AREA_M = np.where(_INSIDE, _AREA_A, np.float32(1.0)).astype(np.float32)
_TF_ROT = ((13, 15, 26, 6), (17, 29, 16, 24))


def _tf_pair(k1, k2, x0, x1):
    ks = (np.uint32(k1), np.uint32(k2),
          np.uint32(np.uint32(k1) ^ np.uint32(k2) ^ np.uint32(0x1BD11BDA)))
    x0 = (x0 + ks[0]).astype(np.uint32)
    x1 = (x1 + ks[1]).astype(np.uint32)
    for r in range(5):
        for d in _TF_ROT[r % 2]:
            x0 = (x0 + x1).astype(np.uint32)
            x1 = ((x1 << np.uint32(d)) | (x1 >> np.uint32(32 - d))).astype(np.uint32)
            x1 = (x0 ^ x1).astype(np.uint32)
        x0 = (x0 + ks[(r + 1) % 3]).astype(np.uint32)
        x1 = (x1 + ks[(r + 2) % 3] + np.uint32(r + 1)).astype(np.uint32)
    return x0, x1


def _tf_uniform(k, n):
    b1, b2 = _tf_pair(k[0], k[1], np.zeros(n, np.uint32),
                      np.arange(n, dtype=np.uint32))
    bits = (b1 ^ b2).astype(np.uint32)
    f = ((bits >> np.uint32(9)) | np.uint32(0x3F800000)).view(np.float32)
    return np.maximum(np.float32(0.0), f - np.float32(1.0))


def _tf_fold_in(k, data):
    b0, b1 = _tf_pair(k[0], k[1],
                      np.array([(data >> 32) & 0xFFFFFFFF], np.uint32),
                      np.array([data & 0xFFFFFFFF], np.uint32))
    return b0[0], b1[0]


_SKEY = (np.uint32(0), np.uint32(42))
_RND_P = _tf_uniform(_SKEY, _A).reshape(_R, _C)
_RND_N = _tf_uniform(_tf_fold_in(_SKEY, 1), _A).reshape(_R, _C)
_M_P = np.round(_RND_P * np.float32(2.0 ** 23)).astype(np.int32)
_M_N = np.round(_RND_N * np.float32(2.0 ** 23)).astype(np.int32)
assert np.array_equal(_M_P.astype(np.float32) * np.float32(2.0 ** -23), _RND_P)
assert np.array_equal(_M_N.astype(np.float32) * np.float32(2.0 ** -23), _RND_N)
_M_PF = _M_P.astype(np.float32)
_M_NF = _M_N.astype(np.float32)
_M_TOP = float(1 << 23)
_M_BIG = float(1 << 24)
_UNROLL = 20


def _body(gt_ref, ay1_ref, ax1_ref, ay2_ref, ax2_ref, area_ref, inside_ref,
          rndp_ref, rndn_ref,
          dy_ref, dx_ref, dh_ref, dw_ref, lab_ref):
    inside = inside_ref[...] != 0

    neg_two = jnp.full((_R, _C), -2.0, dtype=jnp.float32)
    zero = jnp.zeros((_R, _C), dtype=jnp.float32)
    zero_i = jnp.zeros((_R, _C), dtype=jnp.int32)

    def one_gt(j, curmax, bestj, eq):
        gy1 = gt_ref[j, 0]
        gx1 = gt_ref[j, 1]
        gy2 = gt_ref[j, 2]
        gx2 = gt_ref[j, 3]
        tl_y = jnp.maximum(ay1_ref[...], gy1)
        tl_x = jnp.maximum(ax1_ref[...], gx1)
        br_y = jnp.minimum(ay2_ref[...], gy2)
        br_x = jnp.minimum(ax2_ref[...], gx2)
        h = jnp.maximum(br_y - tl_y, 0.0)
        w = jnp.maximum(br_x - tl_x, 0.0)
        inter = h * w
        area_g = (gy2 - gy1) * (gx2 - gx1)
        iou_m = inter / ((area_ref[...] + area_g) - inter)
        gmax = jnp.max(iou_m)
        eq = jnp.where(iou_m == gmax, 1, eq)
        upd = iou_m > curmax
        curmax = jnp.where(upd, iou_m, curmax)
        bestj = jnp.where(upd, j, bestj)
        return curmax, bestj, eq

    def gt_step(jj, carry):
        curmax, bestj, eq = carry
        j0 = jj * _UNROLL
        for t in range(_UNROLL):
            curmax, bestj, eq = one_gt(j0 + t, curmax, bestj, eq)
        return curmax, bestj, eq

    curmax, bestj, eq = lax.fori_loop(
        0, 100 // _UNROLL, gt_step, (neg_two, zero_i, zero_i))

    def exp_step(jj, carry):
        by1, bx1, by2, bx2 = carry
        for t in range(10):
            j = jj * 10 + t
            sel = bestj == j
            by1 = jnp.where(sel, gt_ref[j, 0], by1)
            bx1 = jnp.where(sel, gt_ref[j, 1], bx1)
            by2 = jnp.where(sel, gt_ref[j, 2], by2)
            bx2 = jnp.where(sel, gt_ref[j, 3], bx2)
        return by1, bx1, by2, bx2

    by1, bx1, by2, bx2 = lax.fori_loop(
        0, 10, exp_step, (zero, zero, zero, zero))

    bh = by2 - by1
    bw = bx2 - bx1
    bcy = by1 + 0.5 * bh
    bcx = bx1 + 0.5 * bw
    eps = jnp.float32(np.finfo(np.float32).eps)
    ay1 = ay1_ref[...]
    ax1 = ax1_ref[...]
    ay2 = ay2_ref[...]
    ax2 = ax2_ref[...]
    ah = jnp.maximum(ay2 - ay1, eps)
    aw = jnp.maximum(ax2 - ax1, eps)
    acy = ay1 + 0.5 * (ay2 - ay1)
    acx = ax1 + 0.5 * (ax2 - ax1)
    dy = (bcy - acy) / ah
    dx = (bcx - acx) / aw
    dh = jnp.log(bh / ah)
    dw = jnp.log(bw / aw)
    dy_ref[...] = jnp.where(inside, dy, 0.0)
    dx_ref[...] = jnp.where(inside, dx, 0.0)
    dh_ref[...] = jnp.where(inside, dh, 0.0)
    dw_ref[...] = jnp.where(inside, dw, 0.0)

    labels = jnp.full((_R, _C), -1, dtype=jnp.int32)
    labels = jnp.where(inside & (curmax < _NEG_THRES), 0, labels)
    labels = jnp.where(inside & (eq != 0), 1, labels)
    labels = jnp.where(inside & (curmax >= _POS_THRES), 1, labels)

    pos_mask = labels == 1
    neg_mask = labels == 0
    ones_ll = jnp.ones((_C, _C), dtype=jnp.float32)

    def count_vec(pred):
        ind = jnp.where(pred, 1.0, 0.0).astype(jnp.float32)
        psum = jnp.sum(ind.reshape(9, 8, _C), axis=0)
        lane_tot = jnp.dot(psum, ones_ll)
        return jnp.sum(lane_tot, axis=0, keepdims=True)

    pos_count = count_vec(pos_mask)
    neg_count = count_vec(neg_mask)
    m_p = jnp.where(pos_mask, rndp_ref[...], _M_BIG)
    m_n = jnp.where(neg_mask, rndn_ref[...], _M_BIG)
    n_pos_f = jnp.float32(_N_POS)
    n_neg = jnp.where(pos_count < n_pos_f, pos_count, n_pos_f)
    k_p = jnp.full((1, _C), _N_POS, dtype=jnp.float32)
    k_n = jnp.maximum(n_neg, 1.0)

    def level(_, c):
        lo_p, hi_p, lo_n, hi_n = c

        def mids(lo, hi):
            q = jnp.floor((hi - lo) * 0.25)
            return lo + q, lo + 2.0 * q, lo + 3.0 * q

        def psum(m_plane, mid):
            ind = jnp.where(m_plane <= mid, 1.0, 0.0).astype(jnp.float32)
            return jnp.sum(ind.reshape(9, 8, _C), axis=0)

        mp = mids(lo_p, hi_p)
        mn = mids(lo_n, hi_n)
        stacked = jnp.concatenate(
            [psum(m_p, m) for m in mp] + [psum(m_n, m) for m in mn], axis=0)
        tot = jnp.dot(stacked, ones_ll)
        cnt = jnp.sum(tot.reshape(6, 8, _C), axis=1)

        def narrow(lo, hi, m1, m2, m3, c1, c2, c3, k):
            b1 = c1 >= k
            b2 = c2 >= k
            b3 = c3 >= k
            hi = jnp.where(b1, m1, jnp.where(b2, m2, jnp.where(b3, m3, hi)))
            lo = jnp.where(b1, lo,
                           jnp.where(b2, m1 + 1.0,
                                     jnp.where(b3, m2 + 1.0, m3 + 1.0)))
            return lo, hi

        lo_p, hi_p = narrow(lo_p, hi_p, *mp,
                            cnt[0:1], cnt[1:2], cnt[2:3], k_p)
        lo_n, hi_n = narrow(lo_n, hi_n, *mn,
                            cnt[3:4], cnt[4:5], cnt[5:6], k_n)
        return lo_p, hi_p, lo_n, hi_n

    z = jnp.zeros((1, _C), dtype=jnp.float32)
    top = jnp.full((1, _C), _M_TOP, dtype=jnp.float32)
    lo_p, _, lo_n, _ = lax.fori_loop(0, 14, level, (z, top, z, top))

    kill_p = (pos_count > n_pos_f) & pos_mask & (m_p > lo_p)
    labels = jnp.where(kill_p, -1, labels)
    over_neg = neg_count > n_neg
    kill_n = over_neg & (
        ((n_neg > 0) & neg_mask & (m_n > lo_n)) | ((n_neg == 0) & neg_mask)
    )
    labels = jnp.where(kill_n, -1, labels)
    lab_ref[...] = labels


def kernel(image, feature_map, gt_boxes):
    del image, feature_map
    plane = jax.ShapeDtypeStruct((_R, _C), jnp.float32)
    outs = pl.pallas_call(
        _body,
        out_shape=(
            plane, plane, plane, plane,
            jax.ShapeDtypeStruct((_R, _C), jnp.int32),
        ),
        in_specs=[
            pl.BlockSpec(memory_space=pltpu.SMEM),
            pl.BlockSpec(memory_space=pltpu.VMEM),
            pl.BlockSpec(memory_space=pltpu.VMEM),
            pl.BlockSpec(memory_space=pltpu.VMEM),
            pl.BlockSpec(memory_space=pltpu.VMEM),
            pl.BlockSpec(memory_space=pltpu.VMEM),
            pl.BlockSpec(memory_space=pltpu.VMEM),
            pl.BlockSpec(memory_space=pltpu.VMEM),
            pl.BlockSpec(memory_space=pltpu.VMEM),
        ],
    )(
        gt_boxes,
        jnp.asarray(_AY1_M), jnp.asarray(_AX1_M),
        jnp.asarray(_AY2_M), jnp.asarray(_AX2_M),
        jnp.asarray(_AREA_M),
        jnp.asarray(_INSIDE.astype(np.int32)),
        jnp.asarray(_M_PF), jnp.asarray(_M_NF),
    )
    dy, dx, dh, dw, labels = outs
    locs = jnp.stack(
        [dy.reshape(_A), dx.reshape(_A), dh.reshape(_A), dw.reshape(_A)],
        axis=1,
    )
    return locs, labels.reshape(_A), jnp.asarray(_ANCHORS)


# speedup vs baseline: 1.4518x; 1.0135x over previous
import numpy as np
import jax
import jax.numpy as jnp
from jax import lax
from jax.experimental import pallas as pl
from jax.experimental.pallas import tpu as pltpu

_IM_H, _IM_W = 512, 512
_FEAT_H, _FEAT_W = 32, 32
_STRIDE = 16
_POS_THRES, _NEG_THRES = 0.7, 0.3
_N_SAMPLES = 256
_N_POS = _N_SAMPLES // 2
_A = _FEAT_H * _FEAT_W * 9
_R, _C = 72, 128


def _make_anchors():
    ratios = [0.5, 1.0, 2.0]
    scales = [8.0, 16.0, 32.0]
    hs, ws = [], []
    for r in ratios:
        for s in scales:
            hs.append(_STRIDE * s * np.sqrt(r))
            ws.append(_STRIDE * s * np.sqrt(1.0 / r))
    hs = np.array(hs, dtype=np.float64)
    ws = np.array(ws, dtype=np.float64)
    sy = (np.arange(_FEAT_H) + 0.5) * _STRIDE
    sx = (np.arange(_FEAT_W) + 0.5) * _STRIDE
    cy, cx = np.meshgrid(sy, sx, indexing="ij")
    cy = cy.reshape(-1, 1)
    cx = cx.reshape(-1, 1)
    anchors = np.stack(
        [cy - 0.5 * hs, cx - 0.5 * ws, cy + 0.5 * hs, cx + 0.5 * ws], axis=-1
    )
    return anchors.reshape(-1, 4).astype(np.float32)


_ANCHORS = _make_anchors()
_AY1 = _ANCHORS[:, 0].reshape(_R, _C)
_AX1 = _ANCHORS[:, 1].reshape(_R, _C)
_AY2 = _ANCHORS[:, 2].reshape(_R, _C)
_AX2 = _ANCHORS[:, 3].reshape(_R, _C)
_INSIDE = (
    (_ANCHORS[:, 0] >= 0)
    & (_ANCHORS[:, 1] >= 0)
    & (_ANCHORS[:, 2] <= _IM_H)
    & (_ANCHORS[:, 3] <= _IM_W)
).reshape(_R, _C)
_AREA_A = ((_ANCHORS[:, 2] - _ANCHORS[:, 0]) * (_ANCHORS[:, 3] - _ANCHORS[:, 1])
           ).reshape(_R, _C)
_FAR = np.float32(-1.0e6)
_AY1_M = np.where(_INSIDE, _AY1, _FAR)
_AX1_M = np.where(_INSIDE, _AX1, _FAR)
_AY2_M = np.where(_INSIDE, _AY2, _FAR)
_AX2_M = np.where(_INSIDE, _AX2, _FAR)
_AREA_M = np.where(_INSIDE, _AREA_A, np.float32(1.0)).astype(np.float32)
_TF_ROT = ((13, 15, 26, 6), (17, 29, 16, 24))


def _tf_pair(k1, k2, x0, x1):
    ks = (np.uint32(k1), np.uint32(k2),
          np.uint32(np.uint32(k1) ^ np.uint32(k2) ^ np.uint32(0x1BD11BDA)))
    x0 = (x0 + ks[0]).astype(np.uint32)
    x1 = (x1 + ks[1]).astype(np.uint32)
    for r in range(5):
        for d in _TF_ROT[r % 2]:
            x0 = (x0 + x1).astype(np.uint32)
            x1 = ((x1 << np.uint32(d)) | (x1 >> np.uint32(32 - d))).astype(np.uint32)
            x1 = (x0 ^ x1).astype(np.uint32)
        x0 = (x0 + ks[(r + 1) % 3]).astype(np.uint32)
        x1 = (x1 + ks[(r + 2) % 3] + np.uint32(r + 1)).astype(np.uint32)
    return x0, x1


def _tf_uniform(k, n):
    b1, b2 = _tf_pair(k[0], k[1], np.zeros(n, np.uint32),
                      np.arange(n, dtype=np.uint32))
    bits = (b1 ^ b2).astype(np.uint32)
    f = ((bits >> np.uint32(9)) | np.uint32(0x3F800000)).view(np.float32)
    return np.maximum(np.float32(0.0), f - np.float32(1.0))


def _tf_fold_in(k, data):
    b0, b1 = _tf_pair(k[0], k[1],
                      np.array([(data >> 32) & 0xFFFFFFFF], np.uint32),
                      np.array([data & 0xFFFFFFFF], np.uint32))
    return b0[0], b1[0]


_SKEY = (np.uint32(0), np.uint32(42))
_RND_P = _tf_uniform(_SKEY, _A).reshape(_R, _C)
_RND_N = _tf_uniform(_tf_fold_in(_SKEY, 1), _A).reshape(_R, _C)
_M_P = np.round(_RND_P * np.float32(2.0 ** 23)).astype(np.int32)
_M_N = np.round(_RND_N * np.float32(2.0 ** 23)).astype(np.int32)
assert np.array_equal(_M_P.astype(np.float32) * np.float32(2.0 ** -23), _RND_P)
assert np.array_equal(_M_N.astype(np.float32) * np.float32(2.0 ** -23), _RND_N)
_M_PF = _M_P.astype(np.float32)
_M_NF = _M_N.astype(np.float32)
_M_TOP = float(1 << 23)
_M_BIG = float(1 << 24)
_UNROLL = 25


def _body(gt_ref, ay1_ref, ax1_ref, ay2_ref, ax2_ref, area_ref, inside_ref,
          rndp_ref, rndn_ref,
          dy_ref, dx_ref, dh_ref, dw_ref, lab_ref):
    inside = inside_ref[...] != 0

    neg_two = jnp.full((_R, _C), -2.0, dtype=jnp.float32)
    zero = jnp.zeros((_R, _C), dtype=jnp.float32)
    zero_i = jnp.zeros((_R, _C), dtype=jnp.int32)

    def one_gt(j, curmax, bestj, eq):
        gy1 = gt_ref[j, 0]
        gx1 = gt_ref[j, 1]
        gy2 = gt_ref[j, 2]
        gx2 = gt_ref[j, 3]
        tl_y = jnp.maximum(ay1_ref[...], gy1)
        tl_x = jnp.maximum(ax1_ref[...], gx1)
        br_y = jnp.minimum(ay2_ref[...], gy2)
        br_x = jnp.minimum(ax2_ref[...], gx2)
        h = jnp.maximum(br_y - tl_y, 0.0)
        w = jnp.maximum(br_x - tl_x, 0.0)
        inter = h * w
        area_g = (gy2 - gy1) * (gx2 - gx1)
        iou_m = inter / ((area_ref[...] + area_g) - inter)
        gmax = jnp.max(iou_m)
        eq = jnp.where(iou_m == gmax, 1, eq)
        upd = iou_m > curmax
        curmax = jnp.where(upd, iou_m, curmax)
        bestj = jnp.where(upd, j, bestj)
        return curmax, bestj, eq

    def gt_step(jj, carry):
        curmax, bestj, eq = carry
        j0 = jj * _UNROLL
        for t in range(_UNROLL):
            curmax, bestj, eq = one_gt(j0 + t, curmax, bestj, eq)
        return curmax, bestj, eq

    curmax, bestj, eq = lax.fori_loop(
        0, 100 // _UNROLL, gt_step, (neg_two, zero_i, zero_i))

    def exp_step(jj, carry):
        by1, bx1, by2, bx2 = carry
        for t in range(10):
            j = jj * 10 + t
            sel = bestj == j
            by1 = jnp.where(sel, gt_ref[j, 0], by1)
            bx1 = jnp.where(sel, gt_ref[j, 1], bx1)
            by2 = jnp.where(sel, gt_ref[j, 2], by2)
            bx2 = jnp.where(sel, gt_ref[j, 3], bx2)
        return by1, bx1, by2, bx2

    by1, bx1, by2, bx2 = lax.fori_loop(
        0, 10, exp_step, (zero, zero, zero, zero))

    bh = by2 - by1
    bw = bx2 - bx1
    bcy = by1 + 0.5 * bh
    bcx = bx1 + 0.5 * bw
    eps = jnp.float32(np.finfo(np.float32).eps)
    ay1 = ay1_ref[...]
    ax1 = ax1_ref[...]
    ay2 = ay2_ref[...]
    ax2 = ax2_ref[...]
    ah = jnp.maximum(ay2 - ay1, eps)
    aw = jnp.maximum(ax2 - ax1, eps)
    acy = ay1 + 0.5 * (ay2 - ay1)
    acx = ax1 + 0.5 * (ax2 - ax1)
    dy = (bcy - acy) / ah
    dx = (bcx - acx) / aw
    dh = jnp.log(bh / ah)
    dw = jnp.log(bw / aw)
    dy_ref[...] = jnp.where(inside, dy, 0.0)
    dx_ref[...] = jnp.where(inside, dx, 0.0)
    dh_ref[...] = jnp.where(inside, dh, 0.0)
    dw_ref[...] = jnp.where(inside, dw, 0.0)

    labels = jnp.full((_R, _C), -1, dtype=jnp.int32)
    labels = jnp.where(inside & (curmax < _NEG_THRES), 0, labels)
    labels = jnp.where(inside & (eq != 0), 1, labels)
    labels = jnp.where(inside & (curmax >= _POS_THRES), 1, labels)

    pos_mask = labels == 1
    neg_mask = labels == 0
    ones_ll = jnp.ones((_C, _C), dtype=jnp.float32)

    def count_vec(pred):
        ind = jnp.where(pred, 1.0, 0.0).astype(jnp.float32)
        psum = jnp.sum(ind.reshape(9, 8, _C), axis=0)
        lane_tot = jnp.dot(psum, ones_ll)
        return jnp.sum(lane_tot, axis=0, keepdims=True)

    pos_count = count_vec(pos_mask)
    neg_count = count_vec(neg_mask)
    m_p = jnp.where(pos_mask, rndp_ref[...], _M_BIG)
    m_n = jnp.where(neg_mask, rndn_ref[...], _M_BIG)
    n_pos_f = jnp.float32(_N_POS)
    n_neg = jnp.where(pos_count < n_pos_f, pos_count, n_pos_f)
    k_p = jnp.full((1, _C), _N_POS, dtype=jnp.float32)
    k_n = jnp.maximum(n_neg, 1.0)

    def level(_, c):
        lo_p, hi_p, lo_n, hi_n = c

        def mids(lo, hi):
            q = jnp.floor((hi - lo) * 0.25)
            return lo + q, lo + 2.0 * q, lo + 3.0 * q

        def psum(m_plane, mid):
            ind = jnp.where(m_plane <= mid, 1.0, 0.0).astype(jnp.float32)
            return jnp.sum(ind.reshape(9, 8, _C), axis=0)

        mp = mids(lo_p, hi_p)
        mn = mids(lo_n, hi_n)
        stacked = jnp.concatenate(
            [psum(m_p, m) for m in mp] + [psum(m_n, m) for m in mn], axis=0)
        tot = jnp.dot(stacked, ones_ll)
        cnt = jnp.sum(tot.reshape(6, 8, _C), axis=1)

        def narrow(lo, hi, m1, m2, m3, c1, c2, c3, k):
            b1 = c1 >= k
            b2 = c2 >= k
            b3 = c3 >= k
            hi = jnp.where(b1, m1, jnp.where(b2, m2, jnp.where(b3, m3, hi)))
            lo = jnp.where(b1, lo,
                           jnp.where(b2, m1 + 1.0,
                                     jnp.where(b3, m2 + 1.0, m3 + 1.0)))
            return lo, hi

        lo_p, hi_p = narrow(lo_p, hi_p, *mp,
                            cnt[0:1], cnt[1:2], cnt[2:3], k_p)
        lo_n, hi_n = narrow(lo_n, hi_n, *mn,
                            cnt[3:4], cnt[4:5], cnt[5:6], k_n)
        return lo_p, hi_p, lo_n, hi_n

    z = jnp.zeros((1, _C), dtype=jnp.float32)
    top = jnp.full((1, _C), _M_TOP, dtype=jnp.float32)
    lo_p, _, lo_n, _ = lax.fori_loop(0, 14, level, (z, top, z, top))

    kill_p = (pos_count > n_pos_f) & pos_mask & (m_p > lo_p)
    labels = jnp.where(kill_p, -1, labels)
    over_neg = neg_count > n_neg
    kill_n = over_neg & (
        ((n_neg > 0) & neg_mask & (m_n > lo_n)) | ((n_neg == 0) & neg_mask)
    )
    labels = jnp.where(kill_n, -1, labels)
    lab_ref[...] = labels


def kernel(image, feature_map, gt_boxes):
    del image, feature_map
    plane = jax.ShapeDtypeStruct((_R, _C), jnp.float32)
    outs = pl.pallas_call(
        _body,
        out_shape=(
            plane, plane, plane, plane,
            jax.ShapeDtypeStruct((_R, _C), jnp.int32),
        ),
        in_specs=[
            pl.BlockSpec(memory_space=pltpu.SMEM),
            pl.BlockSpec(memory_space=pltpu.VMEM),
            pl.BlockSpec(memory_space=pltpu.VMEM),
            pl.BlockSpec(memory_space=pltpu.VMEM),
            pl.BlockSpec(memory_space=pltpu.VMEM),
            pl.BlockSpec(memory_space=pltpu.VMEM),
            pl.BlockSpec(memory_space=pltpu.VMEM),
            pl.BlockSpec(memory_space=pltpu.VMEM),
            pl.BlockSpec(memory_space=pltpu.VMEM),
        ],
    )(
        gt_boxes,
        jnp.asarray(_AY1_M), jnp.asarray(_AX1_M),
        jnp.asarray(_AY2_M), jnp.asarray(_AX2_M),
        jnp.asarray(_AREA_M),
        jnp.asarray(_INSIDE.astype(np.int32)),
        jnp.asarray(_M_PF), jnp.asarray(_M_NF),
    )
    dy, dx, dh, dw, labels = outs
    locs = jnp.stack(
        [dy.reshape(_A), dx.reshape(_A), dh.reshape(_A), dw.reshape(_A)],
        axis=1,
    )
    return locs, labels.reshape(_A), jnp.asarray(_ANCHORS)
